# baseline (speedup 1.0000x reference)
import functools

import jax
import jax.numpy as jnp
from jax import lax
from jax.experimental import pallas as pl
from jax.experimental.pallas import tpu as pltpu
from jax.experimental.pallas import tpu_sc as plsc

DIM = 16
L = 16
NC, NS = 2, 16
NW = NC * NS
ROWS_PER_CHUNK = 128 // DIM
GCHUNK = 256


def _rsqrt_newton(s):
    i = lax.bitcast_convert_type(s, jnp.int32)
    i = jnp.int32(0x5F3759DF) - lax.shift_right_arithmetic(i, 1)
    y = lax.bitcast_convert_type(i, jnp.float32)
    for _ in range(3):
        y = y * (jnp.float32(1.5) - jnp.float32(0.5) * s * y * y)
    return y


def _make_sc_loss(batch):
    bpw = batch // NW
    nidx = 2 * bpw
    ndma = nidx // GCHUNK
    rows_per_dma = GCHUNK // 2
    ngrp = rows_per_dma // L
    mesh = plsc.VectorSubcoreMesh(core_axis_name="c", subcore_axis_name="s")

    @functools.partial(
        pl.kernel,
        mesh=mesh,
        out_type=jax.ShapeDtypeStruct((NW, L), jnp.float32),
        scratch_types=[
            pltpu.VMEM((nidx,), jnp.int32),
            pltpu.VMEM((nidx,), jnp.int32),
            pltpu.VMEM((nidx,), jnp.int32),
            pltpu.VMEM((GCHUNK, 128), jnp.float32),
            pltpu.VMEM((GCHUNK, 128), jnp.float32),
            pltpu.VMEM((bpw,), jnp.float32),
            pltpu.VMEM((L,), jnp.float32),
            pltpu.SemaphoreType.DMA,
            pltpu.SemaphoreType.DMA,
        ],
        compiler_params=pltpu.CompilerParams(needs_layout_passes=False),
    )
    def sc_loss(x_hbm, deg_hbm, emb_hbm, out_hbm, idx_v, idxq_v, colb_v,
                buf_a, buf_b, deg_v, acc_v, sem_a, sem_b):
        wid = lax.axis_index("s") * NC + lax.axis_index("c")
        base = wid * bpw
        pltpu.sync_copy(x_hbm.at[pl.ds(2 * base, nidx)], idx_v)
        pltpu.sync_copy(deg_hbm.at[pl.ds(base, bpw)], deg_v)

        def split_body(j, carry):
            v = idx_v[pl.ds(j * L, L)]
            idxq_v[pl.ds(j * L, L)] = lax.shift_right_logical(v, 3)
            colb_v[pl.ds(j * L, L)] = lax.shift_left(
                jnp.bitwise_and(v, jnp.int32(7)), 4
            )
            return carry

        lax.fori_loop(0, nidx // L, split_body, jnp.int32(0))

        bufs = [buf_a, buf_b]
        sems = [sem_a, sem_b]

        def start(c):
            return pltpu.async_copy(
                emb_hbm.at[idxq_v.at[pl.ds(c * GCHUNK, GCHUNK)]],
                bufs[c % 2],
                sems[c % 2],
            )

        lane = lax.iota(jnp.int32, L)

        def chunk_math(c, buf, acc):
            rbase = c * rows_per_dma

            def grp_body(g, acc):
                even = 2 * (g * L + lane)
                odd = even + 1
                col_e = plsc.load_gather(colb_v, [rbase * 2 + even])
                col_o = plsc.load_gather(colb_v, [rbase * 2 + odd])
                ssum = jnp.zeros((L,), jnp.float32)
                for d in range(DIM):
                    a = plsc.load_gather(buf, [even, col_e + d])
                    b = plsc.load_gather(buf, [odd, col_o + d])
                    diff = a - b
                    ssum = ssum + diff * diff
                ssum = jnp.maximum(ssum, jnp.float32(1e-30))
                dist = ssum * _rsqrt_newton(ssum)
                rate = jnp.exp(-dist)
                deg = deg_v[pl.ds(rbase + g * L, L)]
                return acc + deg * dist + rate

            return lax.fori_loop(0, ngrp, grp_body, acc)

        acc = jnp.zeros((L,), jnp.float32)
        dma = start(0)
        for c in range(ndma):
            nxt = start(c + 1) if c + 1 < ndma else None
            dma.wait()
            acc = chunk_math(c, bufs[c % 2], acc)
            dma = nxt

        acc_v[...] = acc
        pltpu.sync_copy(acc_v, out_hbm.at[wid])

    return sc_loss


@jax.jit
def kernel(x, degrees, embs):
    batch = x.shape[0]
    x_flat = x.astype(jnp.int32).reshape(-1)
    emb_chunks = embs.reshape(-1, 128)
    partials = _make_sc_loss(batch)(x_flat, degrees, emb_chunks)
    return jnp.sum(partials)
